# Initial kernel scaffold; baseline (speedup 1.0000x reference)
import jax
import jax.numpy as jnp
from jax.experimental import pallas as pl


def kernel(x, edge_index, W1l, W1r, b1, W2l, W2r, b2, W3l, W3r, b3, Wreg, breg, Wcls, bcls):
    raise NotImplementedError("write your pallas kernel here")


# speedup vs baseline: 7.6754x; 7.6754x over previous
import functools

import jax
import jax.numpy as jnp
from jax import lax
from jax.experimental import pallas as pl
from jax.experimental.pallas import tpu as pltpu
from jax.experimental.pallas import tpu_sc as plsc

N = 10000
E = 320000
NW = 32
CHUNK = 128
C = 79
E_PAD = NW * C * CHUNK
N_ACC = 10112
ROWS_PER_TILE = N_ACC // 16
N8 = N + 8


def _make_sc_agg(dw):
  mesh = plsc.VectorSubcoreMesh(core_axis_name="c", subcore_axis_name="s")

  def body(y_hbm, src_hbm, dst_hbm, z_hbm, out_hbm, src_v, dst_v, rows_v,
           acc, sem):
    c = lax.axis_index("c")
    s = lax.axis_index("s")
    wid = s * 2 + c
    r0 = s * ROWS_PER_TILE
    pltpu.sync_copy(z_hbm.at[pl.ds(r0, ROWS_PER_TILE)],
                    acc.at[pl.ds(r0, ROWS_PER_TILE)])
    pltpu.sync_copy(src_hbm.at[wid], src_v)
    pltpu.sync_copy(dst_hbm.at[wid], dst_v)
    plsc.subcore_barrier()

    def step(j, carry):
      pltpu.async_copy(y_hbm.at[src_v.at[j]], rows_v, sem).wait()
      pltpu.sync_copy(rows_v, acc.at[dst_v.at[j]], add=True)
      return carry

    lax.fori_loop(0, C, step, 0)
    plsc.subcore_barrier()
    pltpu.sync_copy(acc.at[pl.ds(r0, ROWS_PER_TILE)],
                    out_hbm.at[c, pl.ds(r0, ROWS_PER_TILE)])

  return pl.kernel(
      body,
      out_type=jax.ShapeDtypeStruct((2, N_ACC, dw), jnp.float32),
      mesh=mesh,
      compiler_params=pltpu.CompilerParams(use_tc_tiling_on_sc=False),
      scratch_types=[
          pltpu.VMEM((C, CHUNK), jnp.int32),
          pltpu.VMEM((C, CHUNK), jnp.int32),
          pltpu.VMEM((CHUNK, dw), jnp.float32),
          pltpu.VMEM_SHARED((N_ACC, dw), jnp.float32),
          pltpu.SemaphoreType.DMA,
      ],
  )


_sc_agg_80 = _make_sc_agg(80)
_sc_agg_32 = _make_sc_agg(32)
_sc_agg_16 = _make_sc_agg(16)


def _tc0_body(x_ref, w_ref, o_ref):
  y = jnp.dot(x_ref[...], w_ref[...], preferred_element_type=jnp.float32)
  yp = jnp.concatenate(
      [y, jnp.ones((N, 1), jnp.float32), jnp.zeros((N, 15), jnp.float32)],
      axis=1)
  o_ref[...] = jnp.concatenate([yp, jnp.zeros((8, 80), jnp.float32)], axis=0)


_tc0 = pl.pallas_call(
    _tc0_body, out_shape=jax.ShapeDtypeStruct((N8, 80), jnp.float32))


def _tc1_body(agg_ref, x_ref, wr_ref, b_ref, wl2_ref, h_ref, y2_ref, cnt_ref):
  a = agg_ref[0] + agg_ref[1]
  cnt = jnp.maximum(a[:, 64:65], 1.0)
  mean = a[:, :64] / cnt
  h = jnp.maximum(
      mean + jnp.dot(x_ref[...], wr_ref[...],
                     preferred_element_type=jnp.float32) + b_ref[...], 0.0)
  h_ref[...] = h
  y2 = jnp.dot(h, wl2_ref[...], preferred_element_type=jnp.float32)
  y2_ref[...] = jnp.concatenate([y2, jnp.zeros((8, 32), jnp.float32)], axis=0)
  cnt_ref[...] = cnt


_tc1 = pl.pallas_call(
    _tc1_body,
    out_shape=(
        jax.ShapeDtypeStruct((N, 64), jnp.float32),
        jax.ShapeDtypeStruct((N8, 32), jnp.float32),
        jax.ShapeDtypeStruct((N, 1), jnp.float32),
    ))


def _tc2_body(agg_ref, h1_ref, cnt_ref, wr_ref, b_ref, wl3_ref, h_ref, y3_ref):
  a = agg_ref[0] + agg_ref[1]
  mean = a / cnt_ref[...]
  h = jnp.maximum(
      mean + jnp.dot(h1_ref[...], wr_ref[...],
                     preferred_element_type=jnp.float32) + b_ref[...], 0.0)
  h_ref[...] = h
  y3 = jnp.dot(h, wl3_ref[...], preferred_element_type=jnp.float32)
  y3_ref[...] = jnp.concatenate([y3, jnp.zeros((8, 16), jnp.float32)], axis=0)


_tc2 = pl.pallas_call(
    _tc2_body,
    out_shape=(
        jax.ShapeDtypeStruct((N, 32), jnp.float32),
        jax.ShapeDtypeStruct((N8, 16), jnp.float32),
    ))


def _tc3_body(agg_ref, h2_ref, cnt_ref, wr_ref, b_ref, wh_ref, bh_ref, o_ref):
  a = agg_ref[0] + agg_ref[1]
  mean = a / cnt_ref[...]
  h = jnp.maximum(
      mean + jnp.dot(h2_ref[...], wr_ref[...],
                     preferred_element_type=jnp.float32) + b_ref[...], 0.0)
  o_ref[...] = jnp.dot(
      h, wh_ref[...], preferred_element_type=jnp.float32) + bh_ref[...]


_tc3 = pl.pallas_call(
    _tc3_body, out_shape=jax.ShapeDtypeStruct((N, 2), jnp.float32))


@jax.jit
def _run(x, edge_index, W1l, W1r, b1, W2l, W2r, b2, W3l, W3r, b3, Wreg, breg,
         Wcls, bcls):
  ei = edge_index.astype(jnp.int32)
  pad = E_PAD - E
  src = jnp.concatenate([ei[0], jnp.full((pad,), N, jnp.int32)])
  dst = jnp.concatenate([ei[1], jnp.zeros((pad,), jnp.int32)])
  src = src.reshape(NW, C, CHUNK)
  dst = dst.reshape(NW, C, CHUNK)
  z80 = jnp.zeros((N_ACC, 80), jnp.float32)
  z32 = jnp.zeros((N_ACC, 32), jnp.float32)
  z16 = jnp.zeros((N_ACC, 16), jnp.float32)

  y1p = _tc0(x, W1l)
  agg1 = _sc_agg_80(y1p, src, dst, z80)[:, :N]
  h1, y2p, cnt = _tc1(agg1, x, W1r, b1.reshape(1, 64), W2l)
  agg2 = _sc_agg_32(y2p, src, dst, z32)[:, :N]
  h2, y3p = _tc2(agg2, h1, cnt, W2r, b2.reshape(1, 32), W3l)
  agg3 = _sc_agg_16(y3p, src, dst, z16)[:, :N]
  wh = jnp.concatenate([Wreg, Wcls], axis=1)
  bh = jnp.stack([breg[0], bcls[0]]).reshape(1, 2)
  out = _tc3(agg3, h2, cnt, W3r, b3.reshape(1, 16), wh, bh)
  return out[:, 0], out[:, 1]


def kernel(x, edge_index, W1l, W1r, b1, W2l, W2r, b2, W3l, W3r, b3, Wreg,
           breg, Wcls, bcls):
  return _run(x, edge_index, W1l, W1r, b1, W2l, W2r, b2, W3l, W3r, b3, Wreg,
              breg, Wcls, bcls)
